# baseline (speedup 1.0000x reference)
import functools

import jax
import jax.numpy as jnp
from jax import lax
from jax.experimental import pallas as pl
from jax.experimental.pallas import tpu as pltpu
from jax.experimental.pallas import tpu_sc as plsc

N_NODES = 10000
N_EDGES = 320000
NODE_IN = 128
EDGE_IN = 16
HID = 128
OUT_FEATS = 32

_LN_EPS = 1e-5
_INV_SQRT2 = 0.7071067811865476


def _gelu(x):
    return 0.5 * x * (1.0 + lax.erf(x * _INV_SQRT2))


def _proj_node_body(x_ref, w_ref, g_ref, b_ref, o_ref):
    y = jnp.dot(x_ref[...], w_ref[...], preferred_element_type=jnp.float32)
    y = _gelu(y)
    mu = jnp.mean(y, axis=-1, keepdims=True)
    var = jnp.mean((y - mu) ** 2, axis=-1, keepdims=True)
    y = (y - mu) * lax.rsqrt(var + _LN_EPS) * g_ref[...] + b_ref[...]
    o_ref[0] = y[:, :HID // 2]
    o_ref[1] = y[:, HID // 2:]


def _proj_edge_body(x_ref, w_ref, g_ref, b_ref, o_ref):
    y = jnp.dot(x_ref[...], w_ref[...], preferred_element_type=jnp.float32)
    mu = jnp.mean(y, axis=-1, keepdims=True)
    var = jnp.mean((y - mu) ** 2, axis=-1, keepdims=True)
    y = jnp.exp((y - mu) * lax.rsqrt(var + _LN_EPS) * g_ref[...] + b_ref[...])
    o_ref[0] = y[:, :HID // 2]
    o_ref[1] = y[:, HID // 2:]


def _proj_out_body(h_ref, w_ref, g_ref, b_ref, o_ref):
    h = jnp.concatenate([h_ref[0], h_ref[1]], axis=-1)
    y = jnp.dot(h, w_ref[...], preferred_element_type=jnp.float32)
    y = _gelu(y)
    mu = jnp.mean(y, axis=-1, keepdims=True)
    var = jnp.mean((y - mu) ** 2, axis=-1, keepdims=True)
    o_ref[...] = (y - mu) * lax.rsqrt(var + _LN_EPS) * g_ref[...] + b_ref[...]


_NODE_BLK = 1000
_EDGE_BLK = 8192


def _proj_node(x, w, g, b):
    grid = (N_NODES // _NODE_BLK,)
    return pl.pallas_call(
        _proj_node_body,
        grid=grid,
        in_specs=[
            pl.BlockSpec((_NODE_BLK, NODE_IN), lambda i: (i, 0)),
            pl.BlockSpec((NODE_IN, HID), lambda i: (0, 0)),
            pl.BlockSpec((1, HID), lambda i: (0, 0)),
            pl.BlockSpec((1, HID), lambda i: (0, 0)),
        ],
        out_specs=pl.BlockSpec((2, _NODE_BLK, HID // 2), lambda i: (0, i, 0)),
        out_shape=jax.ShapeDtypeStruct((2, N_NODES, HID // 2), jnp.float32),
    )(x, w, g, b)


def _proj_edge(x, w, g, b, n_rows):
    grid = (n_rows // _EDGE_BLK,)
    return pl.pallas_call(
        _proj_edge_body,
        grid=grid,
        in_specs=[
            pl.BlockSpec((_EDGE_BLK, EDGE_IN), lambda i: (i, 0)),
            pl.BlockSpec((EDGE_IN, HID), lambda i: (0, 0)),
            pl.BlockSpec((1, HID), lambda i: (0, 0)),
            pl.BlockSpec((1, HID), lambda i: (0, 0)),
        ],
        out_specs=pl.BlockSpec((2, _EDGE_BLK, HID // 2), lambda i: (0, i, 0)),
        out_shape=jax.ShapeDtypeStruct((2, n_rows, HID // 2), jnp.float32),
    )(x, w, g, b)


def _proj_out(partials, w, g, b):
    grid = (N_NODES // _NODE_BLK,)
    return pl.pallas_call(
        _proj_out_body,
        grid=grid,
        in_specs=[
            pl.BlockSpec((2, _NODE_BLK, HID // 2), lambda i: (0, i, 0)),
            pl.BlockSpec((HID, OUT_FEATS), lambda i: (0, 0)),
            pl.BlockSpec((1, OUT_FEATS), lambda i: (0, 0)),
            pl.BlockSpec((1, OUT_FEATS), lambda i: (0, 0)),
        ],
        out_specs=pl.BlockSpec((_NODE_BLK, OUT_FEATS), lambda i: (i, 0)),
        out_shape=jax.ShapeDtypeStruct((N_NODES, OUT_FEATS), jnp.float32),
    )(partials, w, g, b)


_NC = 2
_NS = 16
_HW = HID // 2
_C = 128
_CPT = 160
_EPT = _CPT * _C
_PAD_EDGES = _NS * _EPT
_DUMMY_NODE = N_NODES
_ROWS_PER_TILE = 632
_PAD_NODES = _ROWS_PER_TILE * _NS
_ZERO_CHUNKS = (128, 128, 128, 128, 120)
_LANES_H = _HW // 16


def _sc_body(hv_hbm, he_hbm, src_hbm, dst_hbm, out_hbm,
             src_v, dst_v, g_v, e_v, m_v,
             sg0, sg1, se0, se1, ss0, ss1, h_sh):
    c = lax.axis_index("c")
    s = lax.axis_index("s")

    pltpu.sync_copy(src_hbm.at[pl.ds(s * _EPT, _EPT)], src_v)
    pltpu.sync_copy(dst_hbm.at[pl.ds(s * _CPT, _CPT)], dst_v)

    half_off = jnp.broadcast_to(c * N_NODES, (16,)).astype(jnp.int32)

    def _adjust(k, carry):
        sl = pl.ds(k * 16, 16)
        src_v[sl] = src_v[sl] + half_off
        return carry

    lax.fori_loop(0, _EPT // 16, _adjust, 0)

    zero = jnp.zeros((16,), jnp.float32)

    def _zero_row(r, carry):
        for j in range(_LANES_H):
            g_v[0, r, pl.ds(j * 16, 16)] = zero
        return carry

    lax.fori_loop(0, _C, _zero_row, 0)
    off = 0
    for n in _ZERO_CHUNKS:
        pltpu.sync_copy(g_v.at[0, pl.ds(0, n)],
                        h_sh.at[pl.ds(s * _ROWS_PER_TILE + off, n)])
        off += n
    plsc.subcore_barrier()

    sg = (sg0, sg1)
    se = (se0, se1)
    ss = (ss0, ss1)

    def _in_copies(cc, b):
        base = cc * _C
        gd = pltpu.make_async_copy(
            hv_hbm.at[src_v.at[pl.ds(base, _C)]], g_v.at[b], sg[b])
        ed = pltpu.make_async_copy(
            he_hbm.at[c, pl.ds(s * _EPT + base, _C)], e_v.at[b], se[b])
        return gd, ed

    def _scatter(cc, b):
        return pltpu.make_async_copy(m_v.at[b], h_sh.at[dst_v.at[cc]], ss[b])

    def _start(cc, b):
        gd, ed = _in_copies(cc, b)
        gd.start()
        ed.start()

    def _step(i2, b):
        cc = 2 * i2 + b
        gd, ed = _in_copies(cc, b)
        gd.wait()
        ed.wait()

        @pl.when(i2 >= 1)
        def _():
            _scatter(cc - 2, b).wait()

        def _mul(r, carry):
            for j in range(_LANES_H):
                sl = pl.ds(j * 16, 16)
                m_v[b, r, sl] = g_v[b, r, sl] * e_v[b, r, sl]
            return carry

        lax.fori_loop(0, _C, _mul, 0)
        _scatter(cc, b).start(add=True)

        @pl.when(i2 <= (_CPT // 2 - 2))
        def _():
            _start(cc + 2, b)

    _start(0, 0)
    _start(1, 1)

    def _loop(i2, carry):
        _step(i2, 0)
        _step(i2, 1)
        return carry

    lax.fori_loop(0, _CPT // 2, _loop, 0)
    for b in range(2):
        _scatter(_CPT - 2 + b, b).wait()

    plsc.subcore_barrier()
    pltpu.sync_copy(h_sh.at[pl.ds(s * _ROWS_PER_TILE, _ROWS_PER_TILE)],
                    out_hbm.at[c, pl.ds(s * _ROWS_PER_TILE, _ROWS_PER_TILE)])


_sc_gather_mul_scatter = functools.partial(
    pl.kernel,
    out_type=jax.ShapeDtypeStruct((_NC, _PAD_NODES, _HW), jnp.float32),
    mesh=plsc.VectorSubcoreMesh(core_axis_name="c", subcore_axis_name="s",
                                num_cores=_NC, num_subcores=_NS),
    compiler_params=pltpu.CompilerParams(use_tc_tiling_on_sc=False),
    scratch_types=[
        pltpu.VMEM((_EPT,), jnp.int32),
        pltpu.VMEM((_CPT, _C), jnp.int32),
        pltpu.VMEM((2, _C, _HW), jnp.float32),
        pltpu.VMEM((2, _C, _HW), jnp.float32),
        pltpu.VMEM((2, _C, _HW), jnp.float32),
        pltpu.SemaphoreType.DMA,
        pltpu.SemaphoreType.DMA,
        pltpu.SemaphoreType.DMA,
        pltpu.SemaphoreType.DMA,
        pltpu.SemaphoreType.DMA,
        pltpu.SemaphoreType.DMA,
        pltpu.VMEM_SHARED((_PAD_NODES, _HW), jnp.float32),
    ],
)(_sc_body)


def kernel(node_feats, edge_feats, edge_index, W_node, g_node, b_node,
           W_edge, g_edge, b_edge, W_out, g_out, b_out):
    hv = _proj_node(node_feats, W_node,
                    g_node.reshape(1, -1), b_node.reshape(1, -1))
    hv2 = hv.reshape(2 * N_NODES, _HW)
    n_pad = _PAD_EDGES - N_EDGES
    ef = jnp.pad(edge_feats, ((0, n_pad), (0, 0)))
    he = _proj_edge(ef, W_edge,
                    g_edge.reshape(1, -1), b_edge.reshape(1, -1), _PAD_EDGES)
    ei = edge_index.astype(jnp.int32)
    src = jnp.concatenate([ei[0], jnp.zeros((n_pad,), jnp.int32)])
    dst = jnp.concatenate([ei[1], jnp.full((n_pad,), _DUMMY_NODE, jnp.int32)])
    dst2d = dst.reshape(_PAD_EDGES // _C, _C)
    partials = _sc_gather_mul_scatter(hv2, he, src, dst2d)
    return _proj_out(partials, W_out,
                     g_out.reshape(1, -1), b_out.reshape(1, -1))


# speedup vs baseline: 1.0102x; 1.0102x over previous
import functools

import jax
import jax.numpy as jnp
from jax import lax
from jax.experimental import pallas as pl
from jax.experimental.pallas import tpu as pltpu
from jax.experimental.pallas import tpu_sc as plsc

N_NODES = 10000
N_EDGES = 320000
NODE_IN = 128
EDGE_IN = 16
HID = 128
OUT_FEATS = 32

_LN_EPS = 1e-5
_INV_SQRT2 = 0.7071067811865476


def _gelu(x):
    return 0.5 * x * (1.0 + lax.erf(x * _INV_SQRT2))


def _proj_node_body(x_ref, w_ref, g_ref, b_ref, o_ref):
    y = jnp.dot(x_ref[...], w_ref[...], preferred_element_type=jnp.float32)
    y = _gelu(y)
    mu = jnp.mean(y, axis=-1, keepdims=True)
    var = jnp.mean((y - mu) ** 2, axis=-1, keepdims=True)
    y = (y - mu) * lax.rsqrt(var + _LN_EPS) * g_ref[...] + b_ref[...]
    o_ref[0] = y[:, :HID // 2]
    o_ref[1] = y[:, HID // 2:]


def _proj_edge_body(x_ref, w_ref, g_ref, b_ref, o_ref):
    y = jnp.dot(x_ref[...], w_ref[...], preferred_element_type=jnp.float32)
    mu = jnp.mean(y, axis=-1, keepdims=True)
    var = jnp.mean((y - mu) ** 2, axis=-1, keepdims=True)
    y = jnp.exp((y - mu) * lax.rsqrt(var + _LN_EPS) * g_ref[...] + b_ref[...])
    o_ref[0] = y[:, :HID // 2]
    o_ref[1] = y[:, HID // 2:]


def _proj_out_body(h_ref, w_ref, g_ref, b_ref, o_ref):
    h = jnp.concatenate([h_ref[0], h_ref[1]], axis=-1)
    y = jnp.dot(h, w_ref[...], preferred_element_type=jnp.float32)
    y = _gelu(y)
    mu = jnp.mean(y, axis=-1, keepdims=True)
    var = jnp.mean((y - mu) ** 2, axis=-1, keepdims=True)
    o_ref[...] = (y - mu) * lax.rsqrt(var + _LN_EPS) * g_ref[...] + b_ref[...]


_NODE_BLK = 1000
_EDGE_BLK = 8000


def _proj_node(x, w, g, b):
    grid = (N_NODES // _NODE_BLK,)
    return pl.pallas_call(
        _proj_node_body,
        grid=grid,
        in_specs=[
            pl.BlockSpec((_NODE_BLK, NODE_IN), lambda i: (i, 0)),
            pl.BlockSpec((NODE_IN, HID), lambda i: (0, 0)),
            pl.BlockSpec((1, HID), lambda i: (0, 0)),
            pl.BlockSpec((1, HID), lambda i: (0, 0)),
        ],
        out_specs=pl.BlockSpec((2, _NODE_BLK, HID // 2), lambda i: (0, i, 0)),
        out_shape=jax.ShapeDtypeStruct((2, N_NODES, HID // 2), jnp.float32),
    )(x, w, g, b)


def _proj_edge(x, w, g, b, n_pad_rows):
    grid = (N_EDGES // _EDGE_BLK,)
    return pl.pallas_call(
        _proj_edge_body,
        grid=grid,
        in_specs=[
            pl.BlockSpec((_EDGE_BLK, EDGE_IN), lambda i: (i, 0)),
            pl.BlockSpec((EDGE_IN, HID), lambda i: (0, 0)),
            pl.BlockSpec((1, HID), lambda i: (0, 0)),
            pl.BlockSpec((1, HID), lambda i: (0, 0)),
        ],
        out_specs=pl.BlockSpec((2, _EDGE_BLK, HID // 2), lambda i: (0, i, 0)),
        out_shape=jax.ShapeDtypeStruct((2, n_pad_rows, HID // 2), jnp.float32),
    )(x, w, g, b)


def _proj_out(partials, w, g, b):
    grid = (N_NODES // _NODE_BLK,)
    return pl.pallas_call(
        _proj_out_body,
        grid=grid,
        in_specs=[
            pl.BlockSpec((2, _NODE_BLK, HID // 2), lambda i: (0, i, 0)),
            pl.BlockSpec((HID, OUT_FEATS), lambda i: (0, 0)),
            pl.BlockSpec((1, OUT_FEATS), lambda i: (0, 0)),
            pl.BlockSpec((1, OUT_FEATS), lambda i: (0, 0)),
        ],
        out_specs=pl.BlockSpec((_NODE_BLK, OUT_FEATS), lambda i: (i, 0)),
        out_shape=jax.ShapeDtypeStruct((N_NODES, OUT_FEATS), jnp.float32),
    )(partials, w, g, b)


_NC = 2
_NS = 16
_HW = HID // 2
_C = 128
_CPT = 160
_EPT = _CPT * _C
_PAD_EDGES = _NS * _EPT
_DUMMY_NODE = N_NODES
_ROWS_PER_TILE = 632
_PAD_NODES = _ROWS_PER_TILE * _NS
_ZERO_CHUNKS = (128, 128, 128, 128, 120)
_LANES_H = _HW // 16


def _sc_body(hv_hbm, he_hbm, src_hbm, dst_hbm, out_hbm,
             src_v, dst_v, g_v, e_v, m_v,
             sg0, sg1, se0, se1, ss0, ss1, h_sh):
    c = lax.axis_index("c")
    s = lax.axis_index("s")

    pltpu.sync_copy(src_hbm.at[c, pl.ds(s * _EPT, _EPT)], src_v)
    pltpu.sync_copy(dst_hbm.at[pl.ds(s * _CPT, _CPT)], dst_v)

    zero = jnp.zeros((16,), jnp.float32)

    def _zero_row(r, carry):
        for j in range(_LANES_H):
            g_v[0, r, pl.ds(j * 16, 16)] = zero
        return carry

    lax.fori_loop(0, _C, _zero_row, 0)
    off = 0
    for n in _ZERO_CHUNKS:
        pltpu.sync_copy(g_v.at[0, pl.ds(0, n)],
                        h_sh.at[pl.ds(s * _ROWS_PER_TILE + off, n)])
        off += n
    plsc.subcore_barrier()

    sg = (sg0, sg1)
    se = (se0, se1)
    ss = (ss0, ss1)

    def _in_copies(cc, b):
        base = cc * _C
        gd = pltpu.make_async_copy(
            hv_hbm.at[src_v.at[pl.ds(base, _C)]], g_v.at[b], sg[b])
        ed = pltpu.make_async_copy(
            he_hbm.at[c, pl.ds(s * _EPT + base, _C)], e_v.at[b], se[b])
        return gd, ed

    def _scatter(cc, b):
        return pltpu.make_async_copy(m_v.at[b], h_sh.at[dst_v.at[cc]], ss[b])

    def _start(cc, b):
        gd, ed = _in_copies(cc, b)
        gd.start()
        ed.start()

    def _step(i2, b):
        cc = 2 * i2 + b
        gd, ed = _in_copies(cc, b)
        gd.wait()
        ed.wait()

        @pl.when(i2 >= 1)
        def _():
            _scatter(cc - 2, b).wait()

        @plsc.parallel_loop(0, _C, 1, unroll=4)
        def _mul(r):
            for j in range(_LANES_H):
                sl = pl.ds(j * 16, 16)
                m_v[b, r, sl] = g_v[b, r, sl] * e_v[b, r, sl]

        _scatter(cc, b).start(add=True)

        @pl.when(i2 <= (_CPT // 2 - 2))
        def _():
            _start(cc + 2, b)

    _start(0, 0)
    _start(1, 1)

    def _loop(i2, carry):
        _step(i2, 0)
        _step(i2, 1)
        return carry

    lax.fori_loop(0, _CPT // 2, _loop, 0)
    for b in range(2):
        _scatter(_CPT - 2 + b, b).wait()

    plsc.subcore_barrier()
    pltpu.sync_copy(h_sh.at[pl.ds(s * _ROWS_PER_TILE, _ROWS_PER_TILE)],
                    out_hbm.at[c, pl.ds(s * _ROWS_PER_TILE, _ROWS_PER_TILE)])


_sc_gather_mul_scatter = functools.partial(
    pl.kernel,
    out_type=jax.ShapeDtypeStruct((_NC, _PAD_NODES, _HW), jnp.float32),
    mesh=plsc.VectorSubcoreMesh(core_axis_name="c", subcore_axis_name="s",
                                num_cores=_NC, num_subcores=_NS),
    compiler_params=pltpu.CompilerParams(use_tc_tiling_on_sc=False),
    scratch_types=[
        pltpu.VMEM((_EPT,), jnp.int32),
        pltpu.VMEM((_CPT, _C), jnp.int32),
        pltpu.VMEM((2, _C, _HW), jnp.float32),
        pltpu.VMEM((2, _C, _HW), jnp.float32),
        pltpu.VMEM((2, _C, _HW), jnp.float32),
        pltpu.SemaphoreType.DMA,
        pltpu.SemaphoreType.DMA,
        pltpu.SemaphoreType.DMA,
        pltpu.SemaphoreType.DMA,
        pltpu.SemaphoreType.DMA,
        pltpu.SemaphoreType.DMA,
        pltpu.VMEM_SHARED((_PAD_NODES, _HW), jnp.float32),
    ],
)(_sc_body)


def kernel(node_feats, edge_feats, edge_index, W_node, g_node, b_node,
           W_edge, g_edge, b_edge, W_out, g_out, b_out):
    hv = _proj_node(node_feats, W_node,
                    g_node.reshape(1, -1), b_node.reshape(1, -1))
    hv2 = hv.reshape(2 * N_NODES, _HW)
    n_pad = _PAD_EDGES - N_EDGES
    he = _proj_edge(edge_feats, W_edge,
                    g_edge.reshape(1, -1), b_edge.reshape(1, -1), _PAD_EDGES)
    ei = edge_index.astype(jnp.int32)
    src = jnp.concatenate([ei[0], jnp.zeros((n_pad,), jnp.int32)])
    src2 = jnp.stack([src, src + N_NODES])
    dst = jnp.concatenate([ei[1], jnp.full((n_pad,), _DUMMY_NODE, jnp.int32)])
    dst2d = dst.reshape(_PAD_EDGES // _C, _C)
    partials = _sc_gather_mul_scatter(hv2, he, src2, dst2d)
    return _proj_out(partials, W_out,
                     g_out.reshape(1, -1), b_out.reshape(1, -1))


# speedup vs baseline: 1.0260x; 1.0157x over previous
import functools

import jax
import jax.numpy as jnp
from jax import lax
from jax.experimental import pallas as pl
from jax.experimental.pallas import tpu as pltpu
from jax.experimental.pallas import tpu_sc as plsc

N_NODES = 10000
N_EDGES = 320000
NODE_IN = 128
EDGE_IN = 16
HID = 128
OUT_FEATS = 32

_LN_EPS = 1e-5
_INV_SQRT2 = 0.7071067811865476


def _gelu(x):
    return 0.5 * x * (1.0 + lax.erf(x * _INV_SQRT2))


def _proj_node_body(x_ref, w_ref, g_ref, b_ref, o_ref):
    y = jnp.dot(x_ref[...], w_ref[...], preferred_element_type=jnp.float32)
    y = _gelu(y)
    mu = jnp.mean(y, axis=-1, keepdims=True)
    var = jnp.mean((y - mu) ** 2, axis=-1, keepdims=True)
    y = (y - mu) * lax.rsqrt(var + _LN_EPS) * g_ref[...] + b_ref[...]
    o_ref[0] = y[:, :HID // 2]
    o_ref[1] = y[:, HID // 2:]


def _proj_edge_body(x_ref, w_ref, g_ref, b_ref, o_ref):
    y = jnp.dot(x_ref[...], w_ref[...], preferred_element_type=jnp.float32)
    mu = jnp.mean(y, axis=-1, keepdims=True)
    var = jnp.mean((y - mu) ** 2, axis=-1, keepdims=True)
    y = jnp.exp((y - mu) * lax.rsqrt(var + _LN_EPS) * g_ref[...] + b_ref[...])
    o_ref[0] = y[:, :HID // 2]
    o_ref[1] = y[:, HID // 2:]


def _proj_out_body(h_ref, w_ref, g_ref, b_ref, o_ref):
    h = jnp.concatenate([h_ref[0], h_ref[1]], axis=-1)
    y = jnp.dot(h, w_ref[...], preferred_element_type=jnp.float32)
    y = _gelu(y)
    mu = jnp.mean(y, axis=-1, keepdims=True)
    var = jnp.mean((y - mu) ** 2, axis=-1, keepdims=True)
    o_ref[...] = (y - mu) * lax.rsqrt(var + _LN_EPS) * g_ref[...] + b_ref[...]


_NODE_BLK = 1000
_EDGE_BLK = 8000


def _proj_node(x, w, g, b):
    grid = (N_NODES // _NODE_BLK,)
    return pl.pallas_call(
        _proj_node_body,
        grid=grid,
        in_specs=[
            pl.BlockSpec((_NODE_BLK, NODE_IN), lambda i: (i, 0)),
            pl.BlockSpec((NODE_IN, HID), lambda i: (0, 0)),
            pl.BlockSpec((1, HID), lambda i: (0, 0)),
            pl.BlockSpec((1, HID), lambda i: (0, 0)),
        ],
        out_specs=pl.BlockSpec((2, _NODE_BLK, HID // 2), lambda i: (0, i, 0)),
        out_shape=jax.ShapeDtypeStruct((2, N_NODES, HID // 2), jnp.float32),
    )(x, w, g, b)


def _proj_edge(x, w, g, b, n_pad_rows):
    grid = (N_EDGES // _EDGE_BLK,)
    return pl.pallas_call(
        _proj_edge_body,
        grid=grid,
        in_specs=[
            pl.BlockSpec((_EDGE_BLK, EDGE_IN), lambda i: (i, 0)),
            pl.BlockSpec((EDGE_IN, HID), lambda i: (0, 0)),
            pl.BlockSpec((1, HID), lambda i: (0, 0)),
            pl.BlockSpec((1, HID), lambda i: (0, 0)),
        ],
        out_specs=pl.BlockSpec((2, _EDGE_BLK, HID // 2), lambda i: (0, i, 0)),
        out_shape=jax.ShapeDtypeStruct((2, n_pad_rows, HID // 2), jnp.float32),
    )(x, w, g, b)


def _proj_out(partials, w, g, b):
    grid = (N_NODES // _NODE_BLK,)
    return pl.pallas_call(
        _proj_out_body,
        grid=grid,
        in_specs=[
            pl.BlockSpec((2, _NODE_BLK, HID // 2), lambda i: (0, i, 0)),
            pl.BlockSpec((HID, OUT_FEATS), lambda i: (0, 0)),
            pl.BlockSpec((1, OUT_FEATS), lambda i: (0, 0)),
            pl.BlockSpec((1, OUT_FEATS), lambda i: (0, 0)),
        ],
        out_specs=pl.BlockSpec((_NODE_BLK, OUT_FEATS), lambda i: (i, 0)),
        out_shape=jax.ShapeDtypeStruct((N_NODES, OUT_FEATS), jnp.float32),
    )(partials, w, g, b)


_NC = 2
_NS = 16
_HW = HID // 2
_C = 128
_CPT = 160
_EPT = _CPT * _C
_PAD_EDGES = _NS * _EPT
_DUMMY_NODE = N_NODES
_ROWS_PER_TILE = 632
_PAD_NODES = _ROWS_PER_TILE * _NS
_ZERO_CHUNKS = (128, 128, 128, 128, 120)
_LANES_H = _HW // 16


def _sc_body(hv_hbm, he_hbm, src_hbm, dst_hbm, out_hbm,
             src_v, dst_v, g_v, e_v,
             sg0, sg1, se0, se1, ss0, ss1, h_sh):
    c = lax.axis_index("c")
    s = lax.axis_index("s")

    pltpu.sync_copy(src_hbm.at[c, pl.ds(s * _EPT, _EPT)], src_v)
    pltpu.sync_copy(dst_hbm.at[pl.ds(s * _CPT, _CPT)], dst_v)

    zero = jnp.zeros((16,), jnp.float32)

    def _zero_row(r, carry):
        for j in range(_LANES_H):
            g_v[0, r, pl.ds(j * 16, 16)] = zero
        return carry

    lax.fori_loop(0, _C, _zero_row, 0)
    off = 0
    for n in _ZERO_CHUNKS:
        pltpu.sync_copy(g_v.at[0, pl.ds(0, n)],
                        h_sh.at[pl.ds(s * _ROWS_PER_TILE + off, n)])
        off += n
    plsc.subcore_barrier()

    sg = (sg0, sg1)
    se = (se0, se1)
    ss = (ss0, ss1)

    def _gather(cc, b):
        return pltpu.make_async_copy(
            hv_hbm.at[src_v.at[pl.ds(cc * _C, _C)]], g_v.at[b], sg[b])

    def _he(cc, b):
        return pltpu.make_async_copy(
            he_hbm.at[c, pl.ds(s * _EPT + cc * _C, _C)], e_v.at[b], se[b])

    def _scatter(cc, b):
        return pltpu.make_async_copy(e_v.at[b], h_sh.at[dst_v.at[cc]], ss[b])

    def _step(i2, b):
        cc = 2 * i2 + b
        _gather(cc, b).wait()
        _he(cc, b).wait()

        @plsc.parallel_loop(0, _C, 1, unroll=4)
        def _mul(r):
            for j in range(_LANES_H):
                sl = pl.ds(j * 16, 16)
                e_v[b, r, sl] = g_v[b, r, sl] * e_v[b, r, sl]

        _scatter(cc, b).start(add=True)
        @pl.when(i2 <= (_CPT // 2 - 2))
        def _():
            _gather(cc + 2, b).start()

        _scatter(cc, b).wait()

        @pl.when(i2 <= (_CPT // 2 - 2))
        def _():
            _he(cc + 2, b).start()

    _gather(0, 0).start()
    _he(0, 0).start()
    _gather(1, 1).start()
    _he(1, 1).start()

    def _loop(i2, carry):
        _step(i2, 0)
        _step(i2, 1)
        return carry

    lax.fori_loop(0, _CPT // 2, _loop, 0)

    plsc.subcore_barrier()
    pltpu.sync_copy(h_sh.at[pl.ds(s * _ROWS_PER_TILE, _ROWS_PER_TILE)],
                    out_hbm.at[c, pl.ds(s * _ROWS_PER_TILE, _ROWS_PER_TILE)])


_sc_gather_mul_scatter = functools.partial(
    pl.kernel,
    out_type=jax.ShapeDtypeStruct((_NC, _PAD_NODES, _HW), jnp.float32),
    mesh=plsc.VectorSubcoreMesh(core_axis_name="c", subcore_axis_name="s",
                                num_cores=_NC, num_subcores=_NS),
    compiler_params=pltpu.CompilerParams(use_tc_tiling_on_sc=False),
    scratch_types=[
        pltpu.VMEM((_EPT,), jnp.int32),
        pltpu.VMEM((_CPT, _C), jnp.int32),
        pltpu.VMEM((2, _C, _HW), jnp.float32),
        pltpu.VMEM((2, _C, _HW), jnp.float32),
        pltpu.SemaphoreType.DMA,
        pltpu.SemaphoreType.DMA,
        pltpu.SemaphoreType.DMA,
        pltpu.SemaphoreType.DMA,
        pltpu.SemaphoreType.DMA,
        pltpu.SemaphoreType.DMA,
        pltpu.VMEM_SHARED((_PAD_NODES, _HW), jnp.float32),
    ],
)(_sc_body)


def kernel(node_feats, edge_feats, edge_index, W_node, g_node, b_node,
           W_edge, g_edge, b_edge, W_out, g_out, b_out):
    hv = _proj_node(node_feats, W_node,
                    g_node.reshape(1, -1), b_node.reshape(1, -1))
    hv2 = hv.reshape(2 * N_NODES, _HW)
    n_pad = _PAD_EDGES - N_EDGES
    he = _proj_edge(edge_feats, W_edge,
                    g_edge.reshape(1, -1), b_edge.reshape(1, -1), _PAD_EDGES)
    ei = edge_index.astype(jnp.int32)
    src = jnp.concatenate([ei[0], jnp.zeros((n_pad,), jnp.int32)])
    src2 = jnp.stack([src, src + N_NODES])
    dst = jnp.concatenate([ei[1], jnp.full((n_pad,), _DUMMY_NODE, jnp.int32)])
    dst2d = dst.reshape(_PAD_EDGES // _C, _C)
    partials = _sc_gather_mul_scatter(hv2, he, src2, dst2d)
    return _proj_out(partials, W_out,
                     g_out.reshape(1, -1), b_out.reshape(1, -1))


# speedup vs baseline: 1.2007x; 1.1702x over previous
import functools

import jax
import jax.numpy as jnp
from jax import lax
from jax.experimental import pallas as pl
from jax.experimental.pallas import tpu as pltpu
from jax.experimental.pallas import tpu_sc as plsc

N_NODES = 10000
N_EDGES = 320000
NODE_IN = 128
EDGE_IN = 16
HID = 128
OUT_FEATS = 32

_LN_EPS = 1e-5
_INV_SQRT2 = 0.7071067811865476


def _gelu(x):
    return 0.5 * x * (1.0 + lax.erf(x * _INV_SQRT2))


def _proj_node_body(x_ref, w_ref, g_ref, b_ref, o_ref):
    y = jnp.dot(x_ref[...], w_ref[...], preferred_element_type=jnp.float32)
    y = _gelu(y)
    mu = jnp.mean(y, axis=-1, keepdims=True)
    var = jnp.mean((y - mu) ** 2, axis=-1, keepdims=True)
    o_ref[...] = (y - mu) * lax.rsqrt(var + _LN_EPS) * g_ref[...] + b_ref[...]


def _proj_edge_body(x_ref, w_ref, g_ref, b_ref, o_ref):
    y = jnp.dot(x_ref[...], w_ref[...], preferred_element_type=jnp.float32)
    mu = jnp.mean(y, axis=-1, keepdims=True)
    var = jnp.mean((y - mu) ** 2, axis=-1, keepdims=True)
    o_ref[...] = jnp.exp((y - mu) * lax.rsqrt(var + _LN_EPS) * g_ref[...] + b_ref[...])


def _proj_out_body(h_ref, w_ref, g_ref, b_ref, o_ref):
    h = h_ref[0] + h_ref[1]
    y = jnp.dot(h, w_ref[...], preferred_element_type=jnp.float32)
    y = _gelu(y)
    mu = jnp.mean(y, axis=-1, keepdims=True)
    var = jnp.mean((y - mu) ** 2, axis=-1, keepdims=True)
    o_ref[...] = (y - mu) * lax.rsqrt(var + _LN_EPS) * g_ref[...] + b_ref[...]


_NODE_BLK = 1000
_EDGE_BLK = 8000


def _proj_node(x, w, g, b):
    grid = (N_NODES // _NODE_BLK,)
    return pl.pallas_call(
        _proj_node_body,
        grid=grid,
        in_specs=[
            pl.BlockSpec((_NODE_BLK, NODE_IN), lambda i: (i, 0)),
            pl.BlockSpec((NODE_IN, HID), lambda i: (0, 0)),
            pl.BlockSpec((1, HID), lambda i: (0, 0)),
            pl.BlockSpec((1, HID), lambda i: (0, 0)),
        ],
        out_specs=pl.BlockSpec((_NODE_BLK, HID), lambda i: (i, 0)),
        out_shape=jax.ShapeDtypeStruct((N_NODES, HID), jnp.float32),
    )(x, w, g, b)


def _proj_edge(x, w, g, b, n_pad_rows):
    grid = (N_EDGES // _EDGE_BLK,)
    return pl.pallas_call(
        _proj_edge_body,
        grid=grid,
        in_specs=[
            pl.BlockSpec((_EDGE_BLK, EDGE_IN), lambda i: (i, 0)),
            pl.BlockSpec((EDGE_IN, HID), lambda i: (0, 0)),
            pl.BlockSpec((1, HID), lambda i: (0, 0)),
            pl.BlockSpec((1, HID), lambda i: (0, 0)),
        ],
        out_specs=pl.BlockSpec((_EDGE_BLK, HID), lambda i: (i, 0)),
        out_shape=jax.ShapeDtypeStruct((n_pad_rows, HID), jnp.float32),
    )(x, w, g, b)


def _proj_out(partials, w, g, b):
    grid = (N_NODES // _NODE_BLK,)
    return pl.pallas_call(
        _proj_out_body,
        grid=grid,
        in_specs=[
            pl.BlockSpec((2, _NODE_BLK, HID), lambda i: (0, i, 0)),
            pl.BlockSpec((HID, OUT_FEATS), lambda i: (0, 0)),
            pl.BlockSpec((1, OUT_FEATS), lambda i: (0, 0)),
            pl.BlockSpec((1, OUT_FEATS), lambda i: (0, 0)),
        ],
        out_specs=pl.BlockSpec((_NODE_BLK, OUT_FEATS), lambda i: (i, 0)),
        out_shape=jax.ShapeDtypeStruct((N_NODES, OUT_FEATS), jnp.float32),
    )(partials, w, g, b)


_NC = 2
_NS = 16
_NW = _NC * _NS
_C = 64
_CPT = 160
_EPT = _CPT * _C
_PAD_EDGES = _NW * _EPT
_DUMMY_NODE = N_NODES
_ROWS_PER_TILE = 632
_PAD_NODES = _ROWS_PER_TILE * _NS
_ZERO_CHUNKS = (64,) * 9 + (56,)
_LANES = HID // 16


def _sc_body(hv_hbm, he_hbm, idx_hbm, out_hbm,
             idx_v, dst_v, g_v, e_v, m_v,
             sg0, sg1, se0, se1, ss0, ss1, si0, si1, h_sh):
    c = lax.axis_index("c")
    s = lax.axis_index("s")
    w = s * _NC + c
    sg = (sg0, sg1)
    se = (se0, se1)
    ss = (ss0, ss1)
    si = (si0, si1)

    zero = jnp.zeros((16,), jnp.float32)

    def _zero_row(r, carry):
        for j in range(_LANES):
            g_v[0, r, pl.ds(j * 16, 16)] = zero
        return carry

    lax.fori_loop(0, _C, _zero_row, 0)
    off = 0
    for n in _ZERO_CHUNKS:
        pltpu.sync_copy(g_v.at[0, pl.ds(0, n)],
                        h_sh.at[pl.ds(s * _ROWS_PER_TILE + off, n)])
        off += n
    plsc.subcore_barrier()

    def _idx(cc, b):
        return pltpu.make_async_copy(
            idx_hbm.at[w * _CPT + cc], idx_v.at[b, 0], si[b])

    def _gather(cc, b):
        del cc
        return pltpu.make_async_copy(
            hv_hbm.at[idx_v.at[b, 0, pl.ds(0, _C)]], g_v.at[b], sg[b])

    def _he(cc, b):
        return pltpu.make_async_copy(
            he_hbm.at[pl.ds(w * _EPT + cc * _C, _C)], e_v.at[b], se[b])

    def _scatter(cc, b):
        del cc
        return pltpu.make_async_copy(m_v.at[b], h_sh.at[dst_v.at[b, 0]], ss[b])

    def _extract_dst(b):
        for j in range(_C // 16):
            sl = pl.ds(j * 16, 16)
            dst_v[b, 0, sl] = idx_v[b, 0, pl.ds(_C + j * 16, 16)]

    def _step(i2, b):
        cc = 2 * i2 + b
        _gather(cc, b).wait()
        _he(cc, b).wait()

        @pl.when(i2 >= 1)
        def _():
            _scatter(cc - 2, b).wait()

        _extract_dst(b)

        @pl.when(i2 <= (_CPT // 2 - 2))
        def _():
            _idx(cc + 2, b).start()

        @plsc.parallel_loop(0, _C, 1, unroll=2)
        def _mul(r):
            for j in range(_LANES):
                sl = pl.ds(j * 16, 16)
                m_v[b, r, sl] = g_v[b, r, sl] * e_v[b, r, sl]

        _scatter(cc, b).start(add=True)

        @pl.when(i2 <= (_CPT // 2 - 2))
        def _():
            _idx(cc + 2, b).wait()
            _gather(cc + 2, b).start()
            _he(cc + 2, b).start()

    for b in range(2):
        _idx(b, b).start()
        _idx(b, b).wait()
        _gather(b, b).start()
        _he(b, b).start()

    def _loop(i2, carry):
        _step(i2, 0)
        _step(i2, 1)
        return carry

    lax.fori_loop(0, _CPT // 2, _loop, 0)
    for b in range(2):
        _scatter(_CPT - 2 + b, b).wait()

    plsc.subcore_barrier()
    pltpu.sync_copy(h_sh.at[pl.ds(s * _ROWS_PER_TILE, _ROWS_PER_TILE)],
                    out_hbm.at[c, pl.ds(s * _ROWS_PER_TILE, _ROWS_PER_TILE)])


_sc_gather_mul_scatter = functools.partial(
    pl.kernel,
    out_type=jax.ShapeDtypeStruct((_NC, _PAD_NODES, HID), jnp.float32),
    mesh=plsc.VectorSubcoreMesh(core_axis_name="c", subcore_axis_name="s",
                                num_cores=_NC, num_subcores=_NS),
    scratch_types=[
        pltpu.VMEM((2, 1, 2 * _C), jnp.int32),
        pltpu.VMEM((2, 1, _C), jnp.int32),
        pltpu.VMEM((2, _C, HID), jnp.float32),
        pltpu.VMEM((2, _C, HID), jnp.float32),
        pltpu.VMEM((2, _C, HID), jnp.float32),
        pltpu.SemaphoreType.DMA,
        pltpu.SemaphoreType.DMA,
        pltpu.SemaphoreType.DMA,
        pltpu.SemaphoreType.DMA,
        pltpu.SemaphoreType.DMA,
        pltpu.SemaphoreType.DMA,
        pltpu.SemaphoreType.DMA,
        pltpu.SemaphoreType.DMA,
        pltpu.VMEM_SHARED((_PAD_NODES, HID), jnp.float32),
    ],
)(_sc_body)


def kernel(node_feats, edge_feats, edge_index, W_node, g_node, b_node,
           W_edge, g_edge, b_edge, W_out, g_out, b_out):
    hv = _proj_node(node_feats, W_node,
                    g_node.reshape(1, -1), b_node.reshape(1, -1))
    n_pad = _PAD_EDGES - N_EDGES
    he = _proj_edge(edge_feats, W_edge,
                    g_edge.reshape(1, -1), b_edge.reshape(1, -1), _PAD_EDGES)
    ei = edge_index.astype(jnp.int32)
    src = jnp.concatenate([ei[0], jnp.zeros((n_pad,), jnp.int32)])
    dst = jnp.concatenate([ei[1], jnp.full((n_pad,), _DUMMY_NODE, jnp.int32)])
    comb = jnp.concatenate([src.reshape(-1, _C), dst.reshape(-1, _C)], axis=1)
    partials = _sc_gather_mul_scatter(hv, he, comb)
    return _proj_out(partials, W_out,
                     g_out.reshape(1, -1), b_out.reshape(1, -1))


# speedup vs baseline: 1.2331x; 1.0270x over previous
import functools

import jax
import jax.numpy as jnp
from jax import lax
from jax.experimental import pallas as pl
from jax.experimental.pallas import tpu as pltpu
from jax.experimental.pallas import tpu_sc as plsc

N_NODES = 10000
N_EDGES = 320000
NODE_IN = 128
EDGE_IN = 16
HID = 128
OUT_FEATS = 32

_LN_EPS = 1e-5
_INV_SQRT2 = 0.7071067811865476


def _gelu(x):
    return 0.5 * x * (1.0 + lax.erf(x * _INV_SQRT2))


def _proj_node_body(x_ref, w_ref, g_ref, b_ref, o_ref):
    y = jnp.dot(x_ref[...], w_ref[...], preferred_element_type=jnp.float32)
    y = _gelu(y)
    mu = jnp.mean(y, axis=-1, keepdims=True)
    var = jnp.mean((y - mu) ** 2, axis=-1, keepdims=True)
    o_ref[...] = (y - mu) * lax.rsqrt(var + _LN_EPS) * g_ref[...] + b_ref[...]


def _proj_edge_body(x_ref, w_ref, g_ref, b_ref, o_ref):
    y = jnp.dot(x_ref[...], w_ref[...], preferred_element_type=jnp.float32)
    mu = jnp.mean(y, axis=-1, keepdims=True)
    var = jnp.mean((y - mu) ** 2, axis=-1, keepdims=True)
    o_ref[...] = jnp.exp((y - mu) * lax.rsqrt(var + _LN_EPS) * g_ref[...] + b_ref[...])


def _proj_out_body(h0_ref, h1_ref, w_ref, g_ref, b_ref, o_ref):
    h = h0_ref[...] + h1_ref[...]
    y = jnp.dot(h, w_ref[...], preferred_element_type=jnp.float32)
    y = _gelu(y)
    mu = jnp.mean(y, axis=-1, keepdims=True)
    var = jnp.mean((y - mu) ** 2, axis=-1, keepdims=True)
    o_ref[...] = (y - mu) * lax.rsqrt(var + _LN_EPS) * g_ref[...] + b_ref[...]


_NODE_BLK = 1000
_EDGE_BLK = 8000


def _proj_node(x, w, g, b):
    grid = (N_NODES // _NODE_BLK,)
    return pl.pallas_call(
        _proj_node_body,
        grid=grid,
        in_specs=[
            pl.BlockSpec((_NODE_BLK, NODE_IN), lambda i: (i, 0)),
            pl.BlockSpec((NODE_IN, HID), lambda i: (0, 0)),
            pl.BlockSpec((1, HID), lambda i: (0, 0)),
            pl.BlockSpec((1, HID), lambda i: (0, 0)),
        ],
        out_specs=pl.BlockSpec((_NODE_BLK, HID), lambda i: (i, 0)),
        out_shape=jax.ShapeDtypeStruct((N_NODES, HID), jnp.float32),
    )(x, w, g, b)


def _proj_edge(x, w, g, b, n_pad_rows):
    grid = (N_EDGES // _EDGE_BLK,)
    return pl.pallas_call(
        _proj_edge_body,
        grid=grid,
        in_specs=[
            pl.BlockSpec((_EDGE_BLK, EDGE_IN), lambda i: (i, 0)),
            pl.BlockSpec((EDGE_IN, HID), lambda i: (0, 0)),
            pl.BlockSpec((1, HID), lambda i: (0, 0)),
            pl.BlockSpec((1, HID), lambda i: (0, 0)),
        ],
        out_specs=pl.BlockSpec((_EDGE_BLK, HID), lambda i: (i, 0)),
        out_shape=jax.ShapeDtypeStruct((n_pad_rows, HID), jnp.float32),
    )(x, w, g, b)


def _proj_out(p0, p1, w, g, b):
    grid = (N_NODES // _NODE_BLK,)
    return pl.pallas_call(
        _proj_out_body,
        grid=grid,
        in_specs=[
            pl.BlockSpec((_NODE_BLK, HID), lambda i: (i, 0)),
            pl.BlockSpec((_NODE_BLK, HID), lambda i: (i, 0)),
            pl.BlockSpec((HID, OUT_FEATS), lambda i: (0, 0)),
            pl.BlockSpec((1, OUT_FEATS), lambda i: (0, 0)),
            pl.BlockSpec((1, OUT_FEATS), lambda i: (0, 0)),
        ],
        out_specs=pl.BlockSpec((_NODE_BLK, OUT_FEATS), lambda i: (i, 0)),
        out_shape=jax.ShapeDtypeStruct((N_NODES, OUT_FEATS), jnp.float32),
    )(p0, p1, w, g, b)


_NC = 2
_NS = 16
_NW = _NC * _NS
_C = 64
_CPT = 160
_EPT = _CPT * _C
_PAD_EDGES = _NW * _EPT
_DUMMY_NODE = N_NODES
_ROWS_PER_TILE = 632
_PAD_NODES = _ROWS_PER_TILE * _NS
_ZERO_CHUNKS = (64,) * 9 + (56,)
_LANES = HID // 16


def _sc_body(hv_hbm, he_hbm, idx_hbm, out0_hbm, out1_hbm,
             idx_v, dst_v, g_v, e_v, m_v,
             sg0, sg1, se0, se1, ss0, ss1, si0, si1, h_sh):
    c = lax.axis_index("c")
    s = lax.axis_index("s")
    w = s * _NC + c
    sg = (sg0, sg1)
    se = (se0, se1)
    ss = (ss0, ss1)
    si = (si0, si1)

    zero = jnp.zeros((16,), jnp.float32)

    def _zero_row(r, carry):
        for j in range(_LANES):
            g_v[0, r, pl.ds(j * 16, 16)] = zero
        return carry

    lax.fori_loop(0, _C, _zero_row, 0)
    off = 0
    for n in _ZERO_CHUNKS:
        pltpu.sync_copy(g_v.at[0, pl.ds(0, n)],
                        h_sh.at[pl.ds(s * _ROWS_PER_TILE + off, n)])
        off += n
    plsc.subcore_barrier()

    def _idx(cc, b):
        return pltpu.make_async_copy(
            idx_hbm.at[w * _CPT + cc], idx_v.at[b, 0], si[b])

    def _gather(cc, b):
        del cc
        return pltpu.make_async_copy(
            hv_hbm.at[idx_v.at[b, 0, pl.ds(0, _C)]], g_v.at[b], sg[b])

    def _he(cc, b):
        return pltpu.make_async_copy(
            he_hbm.at[pl.ds(w * _EPT + cc * _C, _C)], e_v.at[b], se[b])

    def _scatter(cc, b):
        del cc
        return pltpu.make_async_copy(m_v.at[b], h_sh.at[dst_v.at[b, 0]], ss[b])

    def _extract_dst(b):
        for j in range(_C // 16):
            sl = pl.ds(j * 16, 16)
            dst_v[b, 0, sl] = idx_v[b, 0, pl.ds(_C + j * 16, 16)]

    def _step(i2, b):
        cc = 2 * i2 + b
        _gather(cc, b).wait()
        _he(cc, b).wait()

        @pl.when(i2 >= 1)
        def _():
            _scatter(cc - 2, b).wait()

        _extract_dst(b)

        @pl.when(i2 <= (_CPT // 2 - 2))
        def _():
            _idx(cc + 2, b).start()

        @plsc.parallel_loop(0, _C, 1, unroll=2)
        def _mul(r):
            for j in range(_LANES):
                sl = pl.ds(j * 16, 16)
                m_v[b, r, sl] = g_v[b, r, sl] * e_v[b, r, sl]

        _scatter(cc, b).start(add=True)

        @pl.when(i2 <= (_CPT // 2 - 2))
        def _():
            _idx(cc + 2, b).wait()
            _gather(cc + 2, b).start()
            _he(cc + 2, b).start()

    for b in range(2):
        _idx(b, b).start()
        _idx(b, b).wait()
        _gather(b, b).start()
        _he(b, b).start()

    def _loop(i2, carry):
        _step(i2, 0)
        _step(i2, 1)
        return carry

    lax.fori_loop(0, _CPT // 2, _loop, 0)
    for b in range(2):
        _scatter(_CPT - 2 + b, b).wait()

    plsc.subcore_barrier()

    @pl.when(c == 0)
    def _():
        pltpu.sync_copy(h_sh.at[pl.ds(s * _ROWS_PER_TILE, _ROWS_PER_TILE)],
                        out0_hbm.at[pl.ds(s * _ROWS_PER_TILE, _ROWS_PER_TILE)])

    @pl.when(c == 1)
    def _():
        pltpu.sync_copy(h_sh.at[pl.ds(s * _ROWS_PER_TILE, _ROWS_PER_TILE)],
                        out1_hbm.at[pl.ds(s * _ROWS_PER_TILE, _ROWS_PER_TILE)])


_sc_gather_mul_scatter = functools.partial(
    pl.kernel,
    out_type=[jax.ShapeDtypeStruct((_PAD_NODES, HID), jnp.float32),
              jax.ShapeDtypeStruct((_PAD_NODES, HID), jnp.float32)],
    mesh=plsc.VectorSubcoreMesh(core_axis_name="c", subcore_axis_name="s",
                                num_cores=_NC, num_subcores=_NS),
    scratch_types=[
        pltpu.VMEM((2, 1, 2 * _C), jnp.int32),
        pltpu.VMEM((2, 1, _C), jnp.int32),
        pltpu.VMEM((2, _C, HID), jnp.float32),
        pltpu.VMEM((2, _C, HID), jnp.float32),
        pltpu.VMEM((2, _C, HID), jnp.float32),
        pltpu.SemaphoreType.DMA,
        pltpu.SemaphoreType.DMA,
        pltpu.SemaphoreType.DMA,
        pltpu.SemaphoreType.DMA,
        pltpu.SemaphoreType.DMA,
        pltpu.SemaphoreType.DMA,
        pltpu.SemaphoreType.DMA,
        pltpu.SemaphoreType.DMA,
        pltpu.VMEM_SHARED((_PAD_NODES, HID), jnp.float32),
    ],
)(_sc_body)


def kernel(node_feats, edge_feats, edge_index, W_node, g_node, b_node,
           W_edge, g_edge, b_edge, W_out, g_out, b_out):
    hv = _proj_node(node_feats, W_node,
                    g_node.reshape(1, -1), b_node.reshape(1, -1))
    n_pad = _PAD_EDGES - N_EDGES
    he = _proj_edge(edge_feats, W_edge,
                    g_edge.reshape(1, -1), b_edge.reshape(1, -1), _PAD_EDGES)
    ei = edge_index.astype(jnp.int32)
    src = jnp.concatenate([ei[0], jnp.zeros((n_pad,), jnp.int32)])
    dst = jnp.concatenate([ei[1], jnp.full((n_pad,), _DUMMY_NODE, jnp.int32)])
    comb = jnp.concatenate([src.reshape(-1, _C), dst.reshape(-1, _C)], axis=1)
    p0, p1 = _sc_gather_mul_scatter(hv, he, comb)
    return _proj_out(p0, p1, W_out,
                     g_out.reshape(1, -1), b_out.reshape(1, -1))


# speedup vs baseline: 1.2339x; 1.0006x over previous
import functools

import jax
import jax.numpy as jnp
from jax import lax
from jax.experimental import pallas as pl
from jax.experimental.pallas import tpu as pltpu
from jax.experimental.pallas import tpu_sc as plsc

N_NODES = 10000
N_EDGES = 320000
NODE_IN = 128
EDGE_IN = 16
HID = 128
OUT_FEATS = 32

_LN_EPS = 1e-5
_INV_SQRT2 = 0.7071067811865476


def _gelu(x):
    return 0.5 * x * (1.0 + lax.erf(x * _INV_SQRT2))


def _proj_node_body(x_ref, w_ref, g_ref, b_ref, o_ref):
    y = jnp.dot(x_ref[...], w_ref[...], preferred_element_type=jnp.float32)
    y = _gelu(y)
    mu = jnp.mean(y, axis=-1, keepdims=True)
    var = jnp.mean((y - mu) ** 2, axis=-1, keepdims=True)
    o_ref[...] = (y - mu) * lax.rsqrt(var + _LN_EPS) * g_ref[...] + b_ref[...]


def _proj_edge_body(x_ref, w_ref, g_ref, b_ref, o_ref):
    y = jnp.dot(x_ref[...], w_ref[...], preferred_element_type=jnp.float32)
    mu = jnp.mean(y, axis=-1, keepdims=True)
    var = jnp.mean((y - mu) ** 2, axis=-1, keepdims=True)
    o_ref[...] = jnp.exp((y - mu) * lax.rsqrt(var + _LN_EPS) * g_ref[...] + b_ref[...])


def _proj_out_body(h0_ref, h1_ref, w_ref, g_ref, b_ref, o_ref):
    h = h0_ref[...] + h1_ref[...]
    y = jnp.dot(h, w_ref[...], preferred_element_type=jnp.float32)
    y = _gelu(y)
    mu = jnp.mean(y, axis=-1, keepdims=True)
    var = jnp.mean((y - mu) ** 2, axis=-1, keepdims=True)
    o_ref[...] = (y - mu) * lax.rsqrt(var + _LN_EPS) * g_ref[...] + b_ref[...]


_NODE_BLK = 1000
_EDGE_BLK = 8000


def _proj_node(x, w, g, b):
    grid = (N_NODES // _NODE_BLK,)
    return pl.pallas_call(
        _proj_node_body,
        grid=grid,
        in_specs=[
            pl.BlockSpec((_NODE_BLK, NODE_IN), lambda i: (i, 0)),
            pl.BlockSpec((NODE_IN, HID), lambda i: (0, 0)),
            pl.BlockSpec((1, HID), lambda i: (0, 0)),
            pl.BlockSpec((1, HID), lambda i: (0, 0)),
        ],
        out_specs=pl.BlockSpec((_NODE_BLK, HID), lambda i: (i, 0)),
        out_shape=jax.ShapeDtypeStruct((N_NODES, HID), jnp.float32),
    )(x, w, g, b)


def _proj_edge(x, w, g, b, n_pad_rows):
    grid = (N_EDGES // _EDGE_BLK,)
    return pl.pallas_call(
        _proj_edge_body,
        grid=grid,
        in_specs=[
            pl.BlockSpec((_EDGE_BLK, EDGE_IN), lambda i: (i, 0)),
            pl.BlockSpec((EDGE_IN, HID), lambda i: (0, 0)),
            pl.BlockSpec((1, HID), lambda i: (0, 0)),
            pl.BlockSpec((1, HID), lambda i: (0, 0)),
        ],
        out_specs=pl.BlockSpec((_EDGE_BLK, HID), lambda i: (i, 0)),
        out_shape=jax.ShapeDtypeStruct((n_pad_rows, HID), jnp.float32),
    )(x, w, g, b)


def _proj_out(p0, p1, w, g, b):
    grid = (N_NODES // _NODE_BLK,)
    return pl.pallas_call(
        _proj_out_body,
        grid=grid,
        in_specs=[
            pl.BlockSpec((_NODE_BLK, HID), lambda i: (i, 0)),
            pl.BlockSpec((_NODE_BLK, HID), lambda i: (i, 0)),
            pl.BlockSpec((HID, OUT_FEATS), lambda i: (0, 0)),
            pl.BlockSpec((1, OUT_FEATS), lambda i: (0, 0)),
            pl.BlockSpec((1, OUT_FEATS), lambda i: (0, 0)),
        ],
        out_specs=pl.BlockSpec((_NODE_BLK, OUT_FEATS), lambda i: (i, 0)),
        out_shape=jax.ShapeDtypeStruct((N_NODES, OUT_FEATS), jnp.float32),
    )(p0, p1, w, g, b)


_NC = 2
_NS = 16
_NW = _NC * _NS
_C = 64
_CPT = 160
_EPT = _CPT * _C
_PAD_EDGES = _NW * _EPT
_DUMMY_NODE = N_NODES
_ROWS_PER_TILE = 632
_PAD_NODES = _ROWS_PER_TILE * _NS
_ZERO_CHUNKS = (64,) * 9 + (56,)
_LANES = HID // 16


def _sc_body(hv_hbm, he_hbm, idx_hbm, out0_hbm, out1_hbm,
             idx_v, dst_v, g_v, e_v, m_v,
             sg0, sg1, se0, se1, ss0, ss1, si0, si1, h_sh):
    c = lax.axis_index("c")
    s = lax.axis_index("s")
    w = s * _NC + c
    sg = (sg0, sg1)
    se = (se0, se1)
    ss = (ss0, ss1)
    si = (si0, si1)

    zero = jnp.zeros((16,), jnp.float32)

    def _zero_row(r, carry):
        for j in range(_LANES):
            g_v[0, r, pl.ds(j * 16, 16)] = zero
        return carry

    lax.fori_loop(0, _C, _zero_row, 0)
    off = 0
    for n in _ZERO_CHUNKS:
        pltpu.sync_copy(g_v.at[0, pl.ds(0, n)],
                        h_sh.at[pl.ds(s * _ROWS_PER_TILE + off, n)])
        off += n
    plsc.subcore_barrier()

    def _idx(cc, b):
        return pltpu.make_async_copy(
            idx_hbm.at[w * _CPT + cc], idx_v.at[b, 0], si[b])

    def _gather(cc, b):
        del cc
        return pltpu.make_async_copy(
            hv_hbm.at[idx_v.at[b, 0, pl.ds(0, _C)]], g_v.at[b], sg[b])

    def _he(cc, b):
        return pltpu.make_async_copy(
            he_hbm.at[pl.ds(w * _EPT + cc * _C, _C)], e_v.at[b], se[b])

    def _scatter(cc, b):
        del cc
        return pltpu.make_async_copy(m_v.at[b], h_sh.at[dst_v.at[b, 0]], ss[b])

    def _extract_dst(b):
        for j in range(_C // 16):
            sl = pl.ds(j * 16, 16)
            dst_v[b, 0, sl] = idx_v[b, 0, pl.ds(_C + j * 16, 16)]

    def _step(i2, b):
        cc = 2 * i2 + b
        _gather(cc, b).wait()
        _he(cc, b).wait()

        @pl.when(i2 >= 1)
        def _():
            _scatter(cc - 2, b).wait()

        _extract_dst(b)

        @pl.when(i2 <= (_CPT // 2 - 2))
        def _():
            _idx(cc + 2, b).start()

        @plsc.parallel_loop(0, _C, 1, unroll=2)
        def _mul(r):
            for j in range(_LANES):
                sl = pl.ds(j * 16, 16)
                m_v[b, r, sl] = g_v[b, r, sl] * e_v[b, r, sl]

        _scatter(cc, b).start(add=True)

        @pl.when(i2 <= (_CPT // 2 - 2))
        def _():
            _idx(cc + 2, b).wait()
            _gather(cc + 2, b).start()
            _he(cc + 2, b).start()

    for b in range(2):
        _idx(b, b).start()
        _idx(b, b).wait()
        _gather(b, b).start()
        _he(b, b).start()

    def _loop(i2, carry):
        _step(i2, 0)
        _step(i2, 1)
        return carry

    lax.fori_loop(0, _CPT // 2, _loop, 0)
    for b in range(2):
        _scatter(_CPT - 2 + b, b).wait()

    plsc.subcore_barrier()

    @pl.when(c == 0)
    def _():
        pltpu.sync_copy(h_sh.at[pl.ds(s * _ROWS_PER_TILE, _ROWS_PER_TILE)],
                        out0_hbm.at[pl.ds(s * _ROWS_PER_TILE, _ROWS_PER_TILE)])

    @pl.when(c == 1)
    def _():
        pltpu.sync_copy(h_sh.at[pl.ds(s * _ROWS_PER_TILE, _ROWS_PER_TILE)],
                        out1_hbm.at[pl.ds(s * _ROWS_PER_TILE, _ROWS_PER_TILE)])


_sc_gather_mul_scatter = functools.partial(
    pl.kernel,
    out_type=[jax.ShapeDtypeStruct((_PAD_NODES, HID), jnp.float32),
              jax.ShapeDtypeStruct((_PAD_NODES, HID), jnp.float32)],
    mesh=plsc.VectorSubcoreMesh(core_axis_name="c", subcore_axis_name="s",
                                num_cores=_NC, num_subcores=_NS),
    scratch_types=[
        pltpu.VMEM((2, 1, 2 * _C), jnp.int32),
        pltpu.VMEM((2, 1, _C), jnp.int32),
        pltpu.VMEM((2, _C, HID), jnp.float32),
        pltpu.VMEM((2, _C, HID), jnp.float32),
        pltpu.VMEM((2, _C, HID), jnp.float32),
        pltpu.SemaphoreType.DMA,
        pltpu.SemaphoreType.DMA,
        pltpu.SemaphoreType.DMA,
        pltpu.SemaphoreType.DMA,
        pltpu.SemaphoreType.DMA,
        pltpu.SemaphoreType.DMA,
        pltpu.SemaphoreType.DMA,
        pltpu.SemaphoreType.DMA,
        pltpu.VMEM_SHARED((_PAD_NODES, HID), jnp.float32),
    ],
)(_sc_body)


def kernel(node_feats, edge_feats, edge_index, W_node, g_node, b_node,
           W_edge, g_edge, b_edge, W_out, g_out, b_out):
    hv = _proj_node(node_feats, W_node,
                    g_node.reshape(1, -1), b_node.reshape(1, -1))
    n_pad = _PAD_EDGES - N_EDGES
    he = _proj_edge(edge_feats, W_edge,
                    g_edge.reshape(1, -1), b_edge.reshape(1, -1), _PAD_EDGES)
    ei = edge_index.astype(jnp.int32)
    src = jnp.concatenate([ei[0], jnp.zeros((n_pad,), jnp.int32)])
    pad_dst = _DUMMY_NODE + (jnp.arange(n_pad, dtype=jnp.int32) % _C)
    dst = jnp.concatenate([ei[1], pad_dst])
    comb = jnp.concatenate([src.reshape(-1, _C), dst.reshape(-1, _C)], axis=1)
    p0, p1 = _sc_gather_mul_scatter(hv, he, comb)
    return _proj_out(p0, p1, W_out,
                     g_out.reshape(1, -1), b_out.reshape(1, -1))


# speedup vs baseline: 1.9483x; 1.5791x over previous
import functools

import jax
import jax.numpy as jnp
from jax import lax
from jax.experimental import pallas as pl
from jax.experimental.pallas import tpu as pltpu
from jax.experimental.pallas import tpu_sc as plsc

N_NODES = 10000
N_EDGES = 320000
NODE_IN = 128
EDGE_IN = 16
HID = 128
OUT_FEATS = 32

_LN_EPS = 1e-5
_INV_SQRT2 = 0.7071067811865476


def _gelu(x):
    return 0.5 * x * (1.0 + lax.erf(x * _INV_SQRT2))


def _proj_node_body(x_ref, w_ref, g_ref, b_ref, o_ref):
    y = jnp.dot(x_ref[...], w_ref[...], preferred_element_type=jnp.float32)
    y = _gelu(y)
    mu = jnp.mean(y, axis=-1, keepdims=True)
    var = jnp.mean((y - mu) ** 2, axis=-1, keepdims=True)
    o_ref[...] = (y - mu) * lax.rsqrt(var + _LN_EPS) * g_ref[...] + b_ref[...]


def _proj_edge_body(x_ref, w_ref, g_ref, b_ref, o_ref):
    y = jnp.dot(x_ref[...], w_ref[...], preferred_element_type=jnp.float32)
    mu = jnp.mean(y, axis=-1, keepdims=True)
    var = jnp.mean((y - mu) ** 2, axis=-1, keepdims=True)
    o_ref[...] = jnp.exp((y - mu) * lax.rsqrt(var + _LN_EPS) * g_ref[...] + b_ref[...])


def _proj_out_body(h0_ref, h1_ref, w_ref, g_ref, b_ref, o_ref):
    h = h0_ref[...] + h1_ref[...]
    y = jnp.dot(h, w_ref[...], preferred_element_type=jnp.float32)
    y = _gelu(y)
    mu = jnp.mean(y, axis=-1, keepdims=True)
    var = jnp.mean((y - mu) ** 2, axis=-1, keepdims=True)
    o_ref[...] = (y - mu) * lax.rsqrt(var + _LN_EPS) * g_ref[...] + b_ref[...]


_NODE_BLK = 1000
_EDGE_BLK = 8192


def _proj_node(x, w, g, b):
    grid = (N_NODES // _NODE_BLK,)
    return pl.pallas_call(
        _proj_node_body,
        grid=grid,
        in_specs=[
            pl.BlockSpec((_NODE_BLK, NODE_IN), lambda i: (i, 0)),
            pl.BlockSpec((NODE_IN, HID), lambda i: (0, 0)),
            pl.BlockSpec((1, HID), lambda i: (0, 0)),
            pl.BlockSpec((1, HID), lambda i: (0, 0)),
        ],
        out_specs=pl.BlockSpec((_NODE_BLK, HID), lambda i: (i, 0)),
        out_shape=jax.ShapeDtypeStruct((N_NODES, HID), jnp.float32),
    )(x, w, g, b)


def _proj_edge(x, w, g, b, n_pad_rows):
    grid = (n_pad_rows // _EDGE_BLK,)
    return pl.pallas_call(
        _proj_edge_body,
        grid=grid,
        in_specs=[
            pl.BlockSpec((_EDGE_BLK, EDGE_IN), lambda i: (i, 0)),
            pl.BlockSpec((EDGE_IN, HID), lambda i: (0, 0)),
            pl.BlockSpec((1, HID), lambda i: (0, 0)),
            pl.BlockSpec((1, HID), lambda i: (0, 0)),
        ],
        out_specs=pl.BlockSpec((_EDGE_BLK, HID), lambda i: (i, 0)),
        out_shape=jax.ShapeDtypeStruct((n_pad_rows, HID), jnp.float32),
    )(x, w, g, b)


def _proj_out(p0, p1, w, g, b):
    grid = (N_NODES // _NODE_BLK,)
    return pl.pallas_call(
        _proj_out_body,
        grid=grid,
        in_specs=[
            pl.BlockSpec((_NODE_BLK, HID), lambda i: (i, 0)),
            pl.BlockSpec((_NODE_BLK, HID), lambda i: (i, 0)),
            pl.BlockSpec((HID, OUT_FEATS), lambda i: (0, 0)),
            pl.BlockSpec((1, OUT_FEATS), lambda i: (0, 0)),
            pl.BlockSpec((1, OUT_FEATS), lambda i: (0, 0)),
        ],
        out_specs=pl.BlockSpec((_NODE_BLK, OUT_FEATS), lambda i: (i, 0)),
        out_shape=jax.ShapeDtypeStruct((N_NODES, OUT_FEATS), jnp.float32),
    )(p0, p1, w, g, b)


_NC = 2
_NS = 16
_NW = _NC * _NS
_C = 64
_CPT = 160
_EPT = _CPT * _C
_PAD_EDGES = _NW * _EPT
_DUMMY_NODE = N_NODES
_ROWS_PER_TILE = 632
_PAD_NODES = _ROWS_PER_TILE * _NS
_ZERO_CHUNKS = (64,) * 9 + (56,)
_LANES = HID // 16


def _sc_body(hv_hbm, he_hbm, idx_hbm, out0_hbm, out1_hbm,
             idx_v, dst_v, g_v, e_v, m_v,
             sg0, sg1, se0, se1, ss0, ss1, si0, si1, h_sh):
    c = lax.axis_index("c")
    s = lax.axis_index("s")
    w = s * _NC + c
    sg = (sg0, sg1)
    se = (se0, se1)
    ss = (ss0, ss1)
    si = (si0, si1)

    zero = jnp.zeros((16,), jnp.float32)

    def _zero_row(r, carry):
        for j in range(_LANES):
            g_v[0, r, pl.ds(j * 16, 16)] = zero
        return carry

    lax.fori_loop(0, _C, _zero_row, 0)
    off = 0
    for n in _ZERO_CHUNKS:
        pltpu.sync_copy(g_v.at[0, pl.ds(0, n)],
                        h_sh.at[pl.ds(s * _ROWS_PER_TILE + off, n)])
        off += n
    plsc.subcore_barrier()

    def _idx(cc, b):
        return pltpu.make_async_copy(
            idx_hbm.at[w * _CPT + cc], idx_v.at[b, 0], si[b])

    def _gather(cc, b):
        del cc
        return pltpu.make_async_copy(
            hv_hbm.at[idx_v.at[b, 0, pl.ds(0, _C)]], g_v.at[b], sg[b])

    def _he(cc, b):
        return pltpu.make_async_copy(
            he_hbm.at[pl.ds(w * _EPT + cc * _C, _C)], e_v.at[b], se[b])

    def _scatter(cc, b):
        del cc
        return pltpu.make_async_copy(m_v.at[b], h_sh.at[dst_v.at[b, 0]], ss[b])

    def _extract_dst(b):
        for j in range(_C // 16):
            sl = pl.ds(j * 16, 16)
            dst_v[b, 0, sl] = idx_v[b, 0, pl.ds(_C + j * 16, 16)]

    def _step(i2, b):
        cc = 2 * i2 + b
        _gather(cc, b).wait()
        _he(cc, b).wait()

        @pl.when(i2 >= 1)
        def _():
            _scatter(cc - 2, b).wait()

        _extract_dst(b)

        @pl.when(i2 <= (_CPT // 2 - 2))
        def _():
            _idx(cc + 2, b).start()

        @plsc.parallel_loop(0, _C, 1, unroll=2)
        def _mul(r):
            for j in range(_LANES):
                sl = pl.ds(j * 16, 16)
                m_v[b, r, sl] = g_v[b, r, sl] * e_v[b, r, sl]

        _scatter(cc, b).start(add=True)

        @pl.when(i2 <= (_CPT // 2 - 2))
        def _():
            _idx(cc + 2, b).wait()
            _gather(cc + 2, b).start()
            _he(cc + 2, b).start()

    for b in range(2):
        _idx(b, b).start()
        _idx(b, b).wait()
        _gather(b, b).start()
        _he(b, b).start()

    def _loop(i2, carry):
        _step(i2, 0)
        _step(i2, 1)
        return carry

    lax.fori_loop(0, _CPT // 2, _loop, 0)
    for b in range(2):
        _scatter(_CPT - 2 + b, b).wait()

    plsc.subcore_barrier()

    @pl.when(c == 0)
    def _():
        pltpu.sync_copy(h_sh.at[pl.ds(s * _ROWS_PER_TILE, _ROWS_PER_TILE)],
                        out0_hbm.at[pl.ds(s * _ROWS_PER_TILE, _ROWS_PER_TILE)])

    @pl.when(c == 1)
    def _():
        pltpu.sync_copy(h_sh.at[pl.ds(s * _ROWS_PER_TILE, _ROWS_PER_TILE)],
                        out1_hbm.at[pl.ds(s * _ROWS_PER_TILE, _ROWS_PER_TILE)])


_sc_gather_mul_scatter = functools.partial(
    pl.kernel,
    out_type=[jax.ShapeDtypeStruct((_PAD_NODES, HID), jnp.float32),
              jax.ShapeDtypeStruct((_PAD_NODES, HID), jnp.float32)],
    mesh=plsc.VectorSubcoreMesh(core_axis_name="c", subcore_axis_name="s",
                                num_cores=_NC, num_subcores=_NS),
    scratch_types=[
        pltpu.VMEM((2, 1, 2 * _C), jnp.int32),
        pltpu.VMEM((2, 1, _C), jnp.int32),
        pltpu.VMEM((2, _C, HID), jnp.float32),
        pltpu.VMEM((2, _C, HID), jnp.float32),
        pltpu.VMEM((2, _C, HID), jnp.float32),
        pltpu.SemaphoreType.DMA,
        pltpu.SemaphoreType.DMA,
        pltpu.SemaphoreType.DMA,
        pltpu.SemaphoreType.DMA,
        pltpu.SemaphoreType.DMA,
        pltpu.SemaphoreType.DMA,
        pltpu.SemaphoreType.DMA,
        pltpu.SemaphoreType.DMA,
        pltpu.VMEM_SHARED((_PAD_NODES, HID), jnp.float32),
    ],
)(_sc_body)


def kernel(node_feats, edge_feats, edge_index, W_node, g_node, b_node,
           W_edge, g_edge, b_edge, W_out, g_out, b_out):
    hv = _proj_node(node_feats, W_node,
                    g_node.reshape(1, -1), b_node.reshape(1, -1))
    n_pad = _PAD_EDGES - N_EDGES
    ef = jnp.pad(edge_feats, ((0, n_pad), (0, 0)))
    he = _proj_edge(ef, W_edge,
                    g_edge.reshape(1, -1), b_edge.reshape(1, -1), _PAD_EDGES)
    ei = edge_index.astype(jnp.int32)
    pad_src = jnp.arange(n_pad, dtype=jnp.int32) % _C
    src = jnp.concatenate([ei[0], pad_src])
    pad_dst = _DUMMY_NODE + pad_src
    dst = jnp.concatenate([ei[1], pad_dst])
    comb = jnp.concatenate([src.reshape(-1, _C), dst.reshape(-1, _C)], axis=1)
    p0, p1 = _sc_gather_mul_scatter(hv, he, comb)
    return _proj_out(p0, p1, W_out,
                     g_out.reshape(1, -1), b_out.reshape(1, -1))


# speedup vs baseline: 2.0547x; 1.0546x over previous
import functools

import jax
import jax.numpy as jnp
from jax import lax
from jax.experimental import pallas as pl
from jax.experimental.pallas import tpu as pltpu
from jax.experimental.pallas import tpu_sc as plsc

N_NODES = 10000
N_EDGES = 320000
NODE_IN = 128
EDGE_IN = 16
HID = 128
OUT_FEATS = 32

_LN_EPS = 1e-5
_INV_SQRT2 = 0.7071067811865476


def _gelu(x):
    return 0.5 * x * (1.0 + lax.erf(x * _INV_SQRT2))


def _proj_node_body(x_ref, w_ref, g_ref, b_ref, o_ref):
    y = jnp.dot(x_ref[...], w_ref[...], preferred_element_type=jnp.float32)
    y = _gelu(y)
    mu = jnp.mean(y, axis=-1, keepdims=True)
    var = jnp.mean((y - mu) ** 2, axis=-1, keepdims=True)
    o_ref[...] = (y - mu) * lax.rsqrt(var + _LN_EPS) * g_ref[...] + b_ref[...]


def _proj_edge_body(x_ref, w_ref, g_ref, b_ref, o_ref):
    y = jnp.dot(x_ref[...], w_ref[...], preferred_element_type=jnp.float32)
    mu = jnp.mean(y, axis=-1, keepdims=True)
    var = jnp.mean((y - mu) ** 2, axis=-1, keepdims=True)
    o_ref[...] = jnp.exp((y - mu) * lax.rsqrt(var + _LN_EPS) * g_ref[...] + b_ref[...])


def _proj_out_body(h0_ref, h1_ref, w_ref, g_ref, b_ref, o_ref):
    h = h0_ref[...] + h1_ref[...]
    y = jnp.dot(h, w_ref[...], preferred_element_type=jnp.float32)
    y = _gelu(y)
    mu = jnp.mean(y, axis=-1, keepdims=True)
    var = jnp.mean((y - mu) ** 2, axis=-1, keepdims=True)
    o_ref[...] = (y - mu) * lax.rsqrt(var + _LN_EPS) * g_ref[...] + b_ref[...]


_NODE_BLK = 1000
_EDGE_BLK = 8000


def _proj_node(x, w, g, b):
    grid = (N_NODES // _NODE_BLK,)
    return pl.pallas_call(
        _proj_node_body,
        grid=grid,
        in_specs=[
            pl.BlockSpec((_NODE_BLK, NODE_IN), lambda i: (i, 0)),
            pl.BlockSpec((NODE_IN, HID), lambda i: (0, 0)),
            pl.BlockSpec((1, HID), lambda i: (0, 0)),
            pl.BlockSpec((1, HID), lambda i: (0, 0)),
        ],
        out_specs=pl.BlockSpec((_NODE_BLK, HID), lambda i: (i, 0)),
        out_shape=jax.ShapeDtypeStruct((N_NODES, HID), jnp.float32),
    )(x, w, g, b)


def _proj_edge(x, w, g, b, n_pad_rows):
    grid = (n_pad_rows // _EDGE_BLK,)
    last = N_EDGES // _EDGE_BLK - 1
    return pl.pallas_call(
        _proj_edge_body,
        grid=grid,
        in_specs=[
            pl.BlockSpec((_EDGE_BLK, EDGE_IN), lambda i: (jnp.minimum(i, last), 0)),
            pl.BlockSpec((EDGE_IN, HID), lambda i: (0, 0)),
            pl.BlockSpec((1, HID), lambda i: (0, 0)),
            pl.BlockSpec((1, HID), lambda i: (0, 0)),
        ],
        out_specs=pl.BlockSpec((_EDGE_BLK, HID), lambda i: (i, 0)),
        out_shape=jax.ShapeDtypeStruct((n_pad_rows, HID), jnp.float32),
    )(x, w, g, b)


def _proj_out(p0, p1, w, g, b):
    grid = (N_NODES // _NODE_BLK,)
    return pl.pallas_call(
        _proj_out_body,
        grid=grid,
        in_specs=[
            pl.BlockSpec((_NODE_BLK, HID), lambda i: (i, 0)),
            pl.BlockSpec((_NODE_BLK, HID), lambda i: (i, 0)),
            pl.BlockSpec((HID, OUT_FEATS), lambda i: (0, 0)),
            pl.BlockSpec((1, OUT_FEATS), lambda i: (0, 0)),
            pl.BlockSpec((1, OUT_FEATS), lambda i: (0, 0)),
        ],
        out_specs=pl.BlockSpec((_NODE_BLK, OUT_FEATS), lambda i: (i, 0)),
        out_shape=jax.ShapeDtypeStruct((N_NODES, OUT_FEATS), jnp.float32),
    )(p0, p1, w, g, b)


_NC = 2
_NS = 16
_NW = _NC * _NS
_C = 64
_CPT = 160
_EPT = _CPT * _C
_PAD_EDGES = _NW * _EPT
_DUMMY_NODE = N_NODES
_ROWS_PER_TILE = 632
_PAD_NODES = _ROWS_PER_TILE * _NS
_ZERO_CHUNKS = (64,) * 9 + (56,)
_LANES = HID // 16


def _sc_body(hv_hbm, he_hbm, idx_hbm, out0_hbm, out1_hbm,
             idx_v, dst_v, g_v, e_v, m_v,
             sg0, sg1, se0, se1, ss0, ss1, si0, si1, h_sh):
    c = lax.axis_index("c")
    s = lax.axis_index("s")
    w = s * _NC + c
    sg = (sg0, sg1)
    se = (se0, se1)
    ss = (ss0, ss1)
    si = (si0, si1)

    zero = jnp.zeros((16,), jnp.float32)

    def _zero_row(r, carry):
        for j in range(_LANES):
            g_v[0, r, pl.ds(j * 16, 16)] = zero
        return carry

    lax.fori_loop(0, _C, _zero_row, 0)
    off = 0
    for n in _ZERO_CHUNKS:
        pltpu.sync_copy(g_v.at[0, pl.ds(0, n)],
                        h_sh.at[pl.ds(s * _ROWS_PER_TILE + off, n)])
        off += n
    plsc.subcore_barrier()

    def _idx(cc, b):
        return pltpu.make_async_copy(
            idx_hbm.at[w * _CPT + cc], idx_v.at[b, 0], si[b])

    def _gather(cc, b):
        del cc
        return pltpu.make_async_copy(
            hv_hbm.at[idx_v.at[b, 0, pl.ds(0, _C)]], g_v.at[b], sg[b])

    def _he(cc, b):
        return pltpu.make_async_copy(
            he_hbm.at[pl.ds(w * _EPT + cc * _C, _C)], e_v.at[b], se[b])

    def _scatter(cc, b):
        del cc
        return pltpu.make_async_copy(m_v.at[b], h_sh.at[dst_v.at[b, 0]], ss[b])

    def _extract_dst(b):
        for j in range(_C // 16):
            sl = pl.ds(j * 16, 16)
            dst_v[b, 0, sl] = idx_v[b, 0, pl.ds(_C + j * 16, 16)]

    def _step(i2, b):
        cc = 2 * i2 + b
        _gather(cc, b).wait()
        _he(cc, b).wait()

        @pl.when(i2 >= 1)
        def _():
            _scatter(cc - 2, b).wait()

        _extract_dst(b)

        @pl.when(i2 <= (_CPT // 2 - 2))
        def _():
            _idx(cc + 2, b).start()

        @plsc.parallel_loop(0, _C, 1, unroll=2)
        def _mul(r):
            for j in range(_LANES):
                sl = pl.ds(j * 16, 16)
                m_v[b, r, sl] = g_v[b, r, sl] * e_v[b, r, sl]

        _scatter(cc, b).start(add=True)

        @pl.when(i2 <= (_CPT // 2 - 2))
        def _():
            _idx(cc + 2, b).wait()
            _gather(cc + 2, b).start()
            _he(cc + 2, b).start()

    for b in range(2):
        _idx(b, b).start()
        _idx(b, b).wait()
        _gather(b, b).start()
        _he(b, b).start()

    def _loop(i2, carry):
        _step(i2, 0)
        _step(i2, 1)
        return carry

    lax.fori_loop(0, _CPT // 2, _loop, 0)
    for b in range(2):
        _scatter(_CPT - 2 + b, b).wait()

    plsc.subcore_barrier()

    @pl.when(c == 0)
    def _():
        pltpu.sync_copy(h_sh.at[pl.ds(s * _ROWS_PER_TILE, _ROWS_PER_TILE)],
                        out0_hbm.at[pl.ds(s * _ROWS_PER_TILE, _ROWS_PER_TILE)])

    @pl.when(c == 1)
    def _():
        pltpu.sync_copy(h_sh.at[pl.ds(s * _ROWS_PER_TILE, _ROWS_PER_TILE)],
                        out1_hbm.at[pl.ds(s * _ROWS_PER_TILE, _ROWS_PER_TILE)])


_sc_gather_mul_scatter = functools.partial(
    pl.kernel,
    out_type=[jax.ShapeDtypeStruct((_PAD_NODES, HID), jnp.float32),
              jax.ShapeDtypeStruct((_PAD_NODES, HID), jnp.float32)],
    mesh=plsc.VectorSubcoreMesh(core_axis_name="c", subcore_axis_name="s",
                                num_cores=_NC, num_subcores=_NS),
    scratch_types=[
        pltpu.VMEM((2, 1, 2 * _C), jnp.int32),
        pltpu.VMEM((2, 1, _C), jnp.int32),
        pltpu.VMEM((2, _C, HID), jnp.float32),
        pltpu.VMEM((2, _C, HID), jnp.float32),
        pltpu.VMEM((2, _C, HID), jnp.float32),
        pltpu.SemaphoreType.DMA,
        pltpu.SemaphoreType.DMA,
        pltpu.SemaphoreType.DMA,
        pltpu.SemaphoreType.DMA,
        pltpu.SemaphoreType.DMA,
        pltpu.SemaphoreType.DMA,
        pltpu.SemaphoreType.DMA,
        pltpu.SemaphoreType.DMA,
        pltpu.VMEM_SHARED((_PAD_NODES, HID), jnp.float32),
    ],
)(_sc_body)


def kernel(node_feats, edge_feats, edge_index, W_node, g_node, b_node,
           W_edge, g_edge, b_edge, W_out, g_out, b_out):
    hv = _proj_node(node_feats, W_node,
                    g_node.reshape(1, -1), b_node.reshape(1, -1))
    n_pad = _PAD_EDGES - N_EDGES
    n_he = -(-_PAD_EDGES // _EDGE_BLK) * _EDGE_BLK
    he = _proj_edge(edge_feats, W_edge,
                    g_edge.reshape(1, -1), b_edge.reshape(1, -1), n_he)
    ei = edge_index.astype(jnp.int32)
    pad_src = jnp.arange(n_pad, dtype=jnp.int32) % _C
    src = jnp.concatenate([ei[0], pad_src])
    pad_dst = _DUMMY_NODE + pad_src
    dst = jnp.concatenate([ei[1], pad_dst])
    comb = jnp.concatenate([src.reshape(-1, _C), dst.reshape(-1, _C)], axis=1)
    p0, p1 = _sc_gather_mul_scatter(hv, he, comb)
    return _proj_out(p0, p1, W_out,
                     g_out.reshape(1, -1), b_out.reshape(1, -1))
